# Initial kernel scaffold; baseline (speedup 1.0000x reference)
import jax
import jax.numpy as jnp
from jax.experimental import pallas as pl


def kernel(x, edge_index, W1, a1_src, a1_dst, b1, W2, a2_src, a2_dst, b2):
    raise NotImplementedError("write your pallas kernel here")


# speedup vs baseline: 35.9631x; 35.9631x over previous
import functools

import jax
import jax.numpy as jnp
from jax import lax
from jax.experimental import pallas as pl
from jax.experimental.pallas import tpu as pltpu
from jax.experimental.pallas import tpu_sc as plsc

L = 16
N_CORES = 2
N_SUB = 16
CH = 128


def _dense_first(x, W1, a1s, a1d):
    n, _ = x.shape
    d = W1.shape[1]

    def body(x_ref, w_ref, s_ref, t_ref, o_ref, ad_ref):
        h = jnp.dot(x_ref[...], w_ref[...], preferred_element_type=jnp.float32)
        asv = jnp.sum(h * s_ref[...], axis=1, keepdims=True)
        adv = jnp.sum(h * t_ref[...], axis=1, keepdims=True)
        o_ref[...] = jnp.concatenate([h, jnp.ones((n, 1), jnp.float32), asv],
                                     axis=1)
        ad_ref[...] = adv

    return pl.pallas_call(
        body,
        out_shape=[jax.ShapeDtypeStruct((n, d + 2), jnp.float32),
                   jax.ShapeDtypeStruct((n, 1), jnp.float32)],
    )(x, W1, a1s, a1d)


def _norm_dense_second(parts, W2, a2s, a2d, b1, n):
    d = W2.shape[0]
    c = W2.shape[1]

    def body(p_ref, w_ref, s_ref, t_ref, b_ref, o_ref, ad_ref):
        acc = p_ref[0, :n] + p_ref[1, :n]
        ssum = acc[:, d:d + 1]
        h = jnp.where(ssum > 0.0, acc[:, :d] / ssum, 0.0) + b_ref[...]
        h = jnp.maximum(h, 0.0)
        h2 = jnp.dot(h, w_ref[...], preferred_element_type=jnp.float32)
        asv = jnp.sum(h2 * s_ref[...], axis=1, keepdims=True)
        adv = jnp.sum(h2 * t_ref[...], axis=1, keepdims=True)
        o_ref[...] = jnp.concatenate([h2, jnp.ones((n, 1), jnp.float32), asv],
                                     axis=1)
        ad_ref[...] = adv

    return pl.pallas_call(
        body,
        out_shape=[jax.ShapeDtypeStruct((n, c + 2), jnp.float32),
                   jax.ShapeDtypeStruct((n, 1), jnp.float32)],
    )(parts, W2, a2s, a2d, b1)


def _norm_logsoftmax(parts, b2, n):
    c = b2.shape[1]

    def body(p_ref, b_ref, o_ref):
        acc = p_ref[0, :n] + p_ref[1, :n]
        ssum = acc[:, c:c + 1]
        o = jnp.where(ssum > 0.0, acc[:, :c] / ssum, 0.0) + b_ref[...]
        m = jnp.max(o, axis=1, keepdims=True)
        z = o - m
        o_ref[...] = z - jnp.log(jnp.sum(jnp.exp(z), axis=1, keepdims=True))

    return pl.pallas_call(
        body,
        out_shape=jax.ShapeDtypeStruct((n, c), jnp.float32),
    )(parts, b2)


@functools.partial(jax.jit, static_argnames=("d", "n", "e"))
def _edge_pass(table, advec, src, dst, *, d, n, e):
    w_cols = d + 1
    n_workers = N_CORES * N_SUB
    rows_per_sub = ((n + N_SUB - 1) // N_SUB + 159) // 160 * 160
    n_pad = N_SUB * rows_per_sub
    n_chunks = e // CH
    per_w = n_chunks // n_workers
    rem = n_chunks % n_workers

    mesh = plsc.VectorSubcoreMesh(core_axis_name="c", subcore_axis_name="s")

    @functools.partial(
        pl.kernel,
        out_type=jax.ShapeDtypeStruct((N_CORES, n_pad, w_cols), jnp.float32),
        mesh=mesh,
        compiler_params=pltpu.CompilerParams(
            needs_layout_passes=False, use_tc_tiling_on_sc=False),
        scratch_types=[
            pltpu.VMEM((n, 1), jnp.float32),
            pltpu.VMEM((CH,), jnp.int32),
            pltpu.VMEM((CH,), jnp.int32),
            pltpu.VMEM((CH, d + 2), jnp.float32),
            pltpu.VMEM((CH, w_cols), jnp.float32),
            pltpu.VMEM((160, w_cols), jnp.float32),
            pltpu.SemaphoreType.DMA,
            pltpu.VMEM_SHARED((n_pad, w_cols), jnp.float32),
        ],
    )
    def k(tab_hbm, ad_hbm, src_hbm, dst_hbm, out_hbm,
          ad_v, si_v, di_v, rv, wv, zb, sem, acc_s):
        cid = lax.axis_index("c")
        sid = lax.axis_index("s")
        wid = sid * N_CORES + cid
        iota = lax.iota(jnp.int32, L)
        zeros = jnp.zeros((L,), jnp.float32)

        def zrow(r, carry):
            ridx = iota + r * L
            for col in range(w_cols):
                plsc.store_scatter(
                    zb, [ridx, jnp.full((L,), col, jnp.int32)], zeros)
            return carry

        lax.fori_loop(0, 10, zrow, 0)

        def zcp(b, carry):
            pltpu.sync_copy(zb, acc_s.at[pl.ds(sid * rows_per_sub + b * 160, 160)])
            return carry

        lax.fori_loop(0, rows_per_sub // 160, zcp, 0)

        pltpu.sync_copy(ad_hbm, ad_v)
        plsc.subcore_barrier()

        def chunk_body(base):
            pltpu.sync_copy(src_hbm.at[pl.ds(base, CH)], si_v)
            pltpu.sync_copy(dst_hbm.at[pl.ds(base, CH)], di_v)
            pltpu.async_copy(tab_hbm.at[si_v], rv, sem).wait()
            for g in range(CH // L):
                ridx = iota + g * L
                dv = di_v[pl.ds(g * L, L)]
                asv = plsc.load_gather(rv, [ridx, jnp.full((L,), d + 1, jnp.int32)])
                adv = plsc.load_gather(ad_v, [dv, jnp.full((L,), 0, jnp.int32)])
                logit = asv + adv
                logit = jnp.where(logit >= 0.0, logit, logit * 0.2)
                w = jnp.exp(logit)
                for col in range(w_cols):
                    cidx = jnp.full((L,), col, jnp.int32)
                    hv = plsc.load_gather(rv, [ridx, cidx])
                    plsc.store_scatter(wv, [ridx, cidx], w * hv)
            pltpu.sync_copy(wv, acc_s.at[di_v], add=True)

        def loop_body(j, carry):
            chunk_body((wid * per_w + j) * CH)
            return carry

        lax.fori_loop(0, per_w, loop_body, 0)

        if rem:
            @pl.when(wid < rem)
            def _():
                chunk_body((n_workers * per_w + wid) * CH)

        plsc.subcore_barrier()

        def ocp(b, carry):
            r0 = sid * rows_per_sub + b * 160
            pltpu.sync_copy(acc_s.at[pl.ds(r0, 160)], zb)
            pltpu.sync_copy(zb, out_hbm.at[cid, pl.ds(r0, 160)])
            return carry

        lax.fori_loop(0, rows_per_sub // 160, ocp, 0)

    return k(table, advec, src, dst)


def kernel(x, edge_index, W1, a1_src, a1_dst, b1, W2, a2_src, a2_dst, b2):
    n = x.shape[0]
    e = edge_index.shape[1]
    d = W1.shape[1]
    c = W2.shape[1]
    src = edge_index[0].astype(jnp.int32)
    dst = edge_index[1].astype(jnp.int32)

    t1, ad1 = _dense_first(x, W1, a1_src.reshape(1, d), a1_dst.reshape(1, d))
    parts1 = _edge_pass(t1, ad1, src, dst, d=d, n=n, e=e)
    t2, ad2 = _norm_dense_second(parts1, W2, a2_src.reshape(1, c),
                                 a2_dst.reshape(1, c), b1.reshape(1, d), n)
    parts2 = _edge_pass(t2, ad2, src, dst, d=c, n=n, e=e)
    return _norm_logsoftmax(parts2, b2.reshape(1, c), n)


# speedup vs baseline: 60.5067x; 1.6825x over previous
import functools

import jax
import jax.numpy as jnp
from jax import lax
from jax.experimental import pallas as pl
from jax.experimental.pallas import tpu as pltpu
from jax.experimental.pallas import tpu_sc as plsc

L = 16
N_CORES = 2
N_SUB = 16
CH = 128


def _dense_first(x, W1, a1s, a1d):
    n, _ = x.shape
    d = W1.shape[1]

    def body(x_ref, w_ref, s_ref, t_ref, o_ref, ad_ref):
        h = jnp.dot(x_ref[...], w_ref[...], preferred_element_type=jnp.float32)
        asv = jnp.sum(h * s_ref[...], axis=1, keepdims=True)
        adv = jnp.sum(h * t_ref[...], axis=1, keepdims=True)
        o_ref[...] = jnp.concatenate([h, jnp.ones((n, 1), jnp.float32), asv],
                                     axis=1)
        ad_ref[...] = adv

    return pl.pallas_call(
        body,
        out_shape=[jax.ShapeDtypeStruct((n, d + 2), jnp.float32),
                   jax.ShapeDtypeStruct((n, 1), jnp.float32)],
    )(x, W1, a1s, a1d)


def _norm_dense_second(parts, W2, a2s, a2d, b1, n):
    d = W2.shape[0]
    c = W2.shape[1]

    def body(p_ref, w_ref, s_ref, t_ref, b_ref, o_ref, ad_ref):
        acc = p_ref[0, :n] + p_ref[1, :n]
        ssum = acc[:, d:d + 1]
        h = jnp.where(ssum > 0.0, acc[:, :d] / ssum, 0.0) + b_ref[...]
        h = jnp.maximum(h, 0.0)
        h2 = jnp.dot(h, w_ref[...], preferred_element_type=jnp.float32)
        asv = jnp.sum(h2 * s_ref[...], axis=1, keepdims=True)
        adv = jnp.sum(h2 * t_ref[...], axis=1, keepdims=True)
        o_ref[...] = jnp.concatenate([h2, jnp.ones((n, 1), jnp.float32), asv],
                                     axis=1)
        ad_ref[...] = adv

    return pl.pallas_call(
        body,
        out_shape=[jax.ShapeDtypeStruct((n, c + 2), jnp.float32),
                   jax.ShapeDtypeStruct((n, 1), jnp.float32)],
    )(parts, W2, a2s, a2d, b1)


def _norm_logsoftmax(parts, b2, n):
    c = b2.shape[1]

    def body(p_ref, b_ref, o_ref):
        acc = p_ref[0, :n] + p_ref[1, :n]
        ssum = acc[:, c:c + 1]
        o = jnp.where(ssum > 0.0, acc[:, :c] / ssum, 0.0) + b_ref[...]
        m = jnp.max(o, axis=1, keepdims=True)
        z = o - m
        o_ref[...] = z - jnp.log(jnp.sum(jnp.exp(z), axis=1, keepdims=True))

    return pl.pallas_call(
        body,
        out_shape=jax.ShapeDtypeStruct((n, c), jnp.float32),
    )(parts, b2)


@functools.partial(jax.jit, static_argnames=("d", "n", "e"))
def _edge_pass(table, advec, src, dst, *, d, n, e):
    w_cols = d + 1
    n_workers = N_CORES * N_SUB
    rows_per_sub = ((n + N_SUB - 1) // N_SUB + 159) // 160 * 160
    n_pad = N_SUB * rows_per_sub
    n_chunks = e // CH
    per_w = n_chunks // n_workers
    rem = n_chunks % n_workers
    per_w_e = per_w * CH
    assert per_w % 6 == 0, "pipeline unrolls by 6"

    mesh = plsc.VectorSubcoreMesh(core_axis_name="c", subcore_axis_name="s")

    @functools.partial(
        pl.kernel,
        out_type=jax.ShapeDtypeStruct((N_CORES, n_pad, w_cols), jnp.float32),
        mesh=mesh,
        compiler_params=pltpu.CompilerParams(
            needs_layout_passes=False, use_tc_tiling_on_sc=False),
        scratch_types=[
            pltpu.VMEM((n, 1), jnp.float32),
            pltpu.VMEM((per_w_e + CH,), jnp.int32),
            pltpu.VMEM((per_w_e + CH,), jnp.int32),
            [pltpu.VMEM((CH,), jnp.int32) for _ in range(2)],
            [pltpu.VMEM((CH,), jnp.int32) for _ in range(3)],
            [pltpu.VMEM((CH, d + 2), jnp.float32) for _ in range(2)],
            [pltpu.VMEM((CH, w_cols), jnp.float32) for _ in range(2)],
            pltpu.VMEM((160, w_cols), jnp.float32),
            [pltpu.SemaphoreType.DMA for _ in range(2)],
            [pltpu.SemaphoreType.DMA for _ in range(2)],
            pltpu.VMEM_SHARED((n_pad, w_cols), jnp.float32),
        ],
    )
    def k(tab_hbm, ad_hbm, src_hbm, dst_hbm, out_hbm,
          ad_v, sb_v, db_v, si, di, rv, wv, zb, gsem, ssem, acc_s):
        cid = lax.axis_index("c")
        sid = lax.axis_index("s")
        wid = sid * N_CORES + cid
        iota = lax.iota(jnp.int32, L)
        zeros = jnp.zeros((L,), jnp.float32)

        def zrow(r, carry):
            ridx = iota + r * L
            for col in range(w_cols):
                plsc.store_scatter(
                    zb, [ridx, jnp.full((L,), col, jnp.int32)], zeros)
            return carry

        lax.fori_loop(0, 10, zrow, 0)

        def zcp(b, carry):
            pltpu.sync_copy(zb, acc_s.at[pl.ds(sid * rows_per_sub + b * 160, 160)])
            return carry

        lax.fori_loop(0, rows_per_sub // 160, zcp, 0)

        pltpu.sync_copy(ad_hbm, ad_v)
        pltpu.sync_copy(src_hbm.at[pl.ds(wid * per_w_e, per_w_e)],
                        sb_v.at[pl.ds(0, per_w_e)])
        pltpu.sync_copy(dst_hbm.at[pl.ds(wid * per_w_e, per_w_e)],
                        db_v.at[pl.ds(0, per_w_e)])
        izeros = jnp.zeros((L,), jnp.int32)
        for g in range(CH // L):
            plsc.store_scatter(sb_v, [iota + per_w_e + g * L], izeros)
            plsc.store_scatter(db_v, [iota + per_w_e + g * L], izeros)
        plsc.subcore_barrier()

        def fill_idx(j, b2, b3):
            for g in range(CH // L):
                ridx = iota + g * L
                sv = plsc.load_gather(sb_v, [ridx + j * CH])
                dv = plsc.load_gather(db_v, [ridx + j * CH])
                plsc.store_scatter(si[b2], [ridx], sv)
                plsc.store_scatter(di[b3], [ridx], dv)

        def start_gather(b2):
            return pltpu.async_copy(tab_hbm.at[si[b2]], rv[b2], gsem[b2])

        def compute(b2in, b2out, b3):
            for g in range(CH // L):
                ridx = iota + g * L
                dv = di[b3][pl.ds(g * L, L)]
                asv = plsc.load_gather(
                    rv[b2in], [ridx, jnp.full((L,), d + 1, jnp.int32)])
                adv = plsc.load_gather(ad_v, [dv, jnp.full((L,), 0, jnp.int32)])
                logit = asv + adv
                logit = jnp.where(logit >= 0.0, logit, logit * 0.2)
                w = jnp.exp(logit)
                for col in range(w_cols):
                    cidx = jnp.full((L,), col, jnp.int32)
                    hv = plsc.load_gather(rv[b2in], [ridx, cidx])
                    plsc.store_scatter(wv[b2out], [ridx, cidx], w * hv)

        def start_scatter(b2, b3):
            return pltpu.async_copy(wv[b2], acc_s.at[di[b3]], ssem[b2],
                                    add=True)

        def wait_gather(b2):
            pltpu.make_async_copy(tab_hbm.at[si[b2]], rv[b2], gsem[b2]).wait()

        def wait_scatter(b2, b3):
            pltpu.make_async_copy(wv[b2], acc_s.at[di[b3]], ssem[b2]).wait()

        def one_iter(k, u, wait_s):
            b2, n2, b3, n3 = u % 2, (u + 1) % 2, u % 3, (u + 1) % 3
            wait_gather(b2)
            if wait_s:
                wait_scatter(b2, b3)
            fill_idx(k + 1, n2, n3)
            start_gather(n2)
            compute(b2, b2, b3)
            start_scatter(b2, b3)

        fill_idx(0, 0, 0)
        start_gather(0)
        for u in range(6):
            one_iter(u, u, u >= 2)

        def loop_body(k6, carry):
            k0 = k6 * 6
            for u in range(6):
                one_iter(k0 + u, u, True)
            return carry

        lax.fori_loop(1, per_w // 6, loop_body, 0)

        wait_gather(per_w % 2)
        wait_scatter((per_w - 2) % 2, (per_w - 2) % 3)
        wait_scatter((per_w - 1) % 2, (per_w - 1) % 3)

        if rem:
            @pl.when(wid < rem)
            def _():
                base = (n_workers * per_w + wid) * CH
                pltpu.sync_copy(src_hbm.at[pl.ds(base, CH)],
                                sb_v.at[pl.ds(0, CH)])
                pltpu.sync_copy(dst_hbm.at[pl.ds(base, CH)],
                                db_v.at[pl.ds(0, CH)])
                fill_idx(0, 0, 0)
                start_gather(0)
                wait_gather(0)
                compute(0, 0, 0)
                start_scatter(0, 0)
                wait_scatter(0, 0)

        plsc.subcore_barrier()

        def ocp(b, carry):
            r0 = sid * rows_per_sub + b * 160
            pltpu.sync_copy(acc_s.at[pl.ds(r0, 160)], zb)
            pltpu.sync_copy(zb, out_hbm.at[cid, pl.ds(r0, 160)])
            return carry

        lax.fori_loop(0, rows_per_sub // 160, ocp, 0)

    return k(table, advec, src, dst)


def kernel(x, edge_index, W1, a1_src, a1_dst, b1, W2, a2_src, a2_dst, b2):
    n = x.shape[0]
    e = edge_index.shape[1]
    d = W1.shape[1]
    c = W2.shape[1]
    src = edge_index[0].astype(jnp.int32)
    dst = edge_index[1].astype(jnp.int32)

    t1, ad1 = _dense_first(x, W1, a1_src.reshape(1, d), a1_dst.reshape(1, d))
    parts1 = _edge_pass(t1, ad1, src, dst, d=d, n=n, e=e)
    t2, ad2 = _norm_dense_second(parts1, W2, a2_src.reshape(1, c),
                                 a2_dst.reshape(1, c), b1.reshape(1, d), n)
    parts2 = _edge_pass(t2, ad2, src, dst, d=c, n=n, e=e)
    return _norm_logsoftmax(parts2, b2.reshape(1, c), n)


# speedup vs baseline: 78.2458x; 1.2932x over previous
import functools

import jax
import jax.numpy as jnp
from jax import lax
from jax.experimental import pallas as pl
from jax.experimental.pallas import tpu as pltpu
from jax.experimental.pallas import tpu_sc as plsc

L = 16
N_CORES = 2
N_SUB = 16
CH = 128


def _dense_first(xT, W1T, a1s, a1d):
    n = xT.shape[1]
    d = W1T.shape[0]

    def body(x_ref, w_ref, s_ref, t_ref, o_ref, ad_ref):
        hT = jnp.dot(w_ref[...], x_ref[...], preferred_element_type=jnp.float32)
        asT = jnp.dot(s_ref[...], hT, preferred_element_type=jnp.float32)
        adT = jnp.dot(t_ref[...], hT, preferred_element_type=jnp.float32)
        o_ref[...] = jnp.concatenate([hT, asT], axis=0)
        ad_ref[...] = adT

    return pl.pallas_call(
        body,
        out_shape=[jax.ShapeDtypeStruct((d + 1, n), jnp.float32),
                   jax.ShapeDtypeStruct((1, n), jnp.float32)],
    )(xT, W1T, a1s, a1d)


def _norm_dense_second(parts, W2tr, a2s, a2d, b1, n):
    c, d = W2tr.shape

    def body(p_ref, w_ref, s_ref, t_ref, b_ref, o_ref, ad_ref):
        acc = p_ref[0, :n] + p_ref[1, :n]
        ssum = acc[:, d:d + 1]
        h = jnp.where(ssum > 0.0, acc[:, :d] / ssum, 0.0) + b_ref[...]
        h = jnp.maximum(h, 0.0)
        h2T = lax.dot_general(w_ref[...], h, (((1,), (1,)), ((), ())),
                              preferred_element_type=jnp.float32)
        asT = jnp.dot(s_ref[...], h2T, preferred_element_type=jnp.float32)
        adT = jnp.dot(t_ref[...], h2T, preferred_element_type=jnp.float32)
        o_ref[...] = jnp.concatenate([h2T, asT], axis=0)
        ad_ref[...] = adT

    return pl.pallas_call(
        body,
        out_shape=[jax.ShapeDtypeStruct((c + 1, n), jnp.float32),
                   jax.ShapeDtypeStruct((1, n), jnp.float32)],
    )(parts, W2tr, a2s, a2d, b1)


def _norm_logsoftmax(parts, b2, n):
    c = b2.shape[1]

    def body(p_ref, b_ref, o_ref):
        acc = p_ref[0, :n] + p_ref[1, :n]
        ssum = acc[:, c:c + 1]
        o = jnp.where(ssum > 0.0, acc[:, :c] / ssum, 0.0) + b_ref[...]
        m = jnp.max(o, axis=1, keepdims=True)
        z = o - m
        o_ref[...] = z - jnp.log(jnp.sum(jnp.exp(z), axis=1, keepdims=True))

    return pl.pallas_call(
        body,
        out_shape=jax.ShapeDtypeStruct((n, c), jnp.float32),
    )(parts, b2)


@functools.partial(jax.jit, static_argnames=("d", "n", "e"))
def _edge_pass(tableT, advec, src, dst, *, d, n, e):
    w_cols = d + 1
    n_workers = N_CORES * N_SUB
    rows_per_sub = ((n + N_SUB - 1) // N_SUB + 159) // 160 * 160
    n_pad = N_SUB * rows_per_sub
    n_chunks = e // CH
    per_w = n_chunks // n_workers
    rem = n_chunks % n_workers
    assert per_w % 4 == 2 and per_w >= 6, "pipeline peels 4 + tail 2"

    mesh = plsc.VectorSubcoreMesh(core_axis_name="c", subcore_axis_name="s")

    @functools.partial(
        pl.kernel,
        out_type=jax.ShapeDtypeStruct((N_CORES, n_pad, w_cols), jnp.float32),
        mesh=mesh,
        compiler_params=pltpu.CompilerParams(
            needs_layout_passes=False, use_tc_tiling_on_sc=False),
        scratch_types=[
            pltpu.VMEM((d + 1, n), jnp.float32),
            [pltpu.VMEM((CH,), jnp.int32) for _ in range(4)],
            [pltpu.VMEM((CH,), jnp.int32) for _ in range(4)],
            [pltpu.VMEM((CH,), jnp.float32) for _ in range(2)],
            [pltpu.VMEM((CH, w_cols), jnp.float32) for _ in range(2)],
            pltpu.VMEM((160, w_cols), jnp.float32),
            pltpu.SemaphoreType.DMA,
            [pltpu.SemaphoreType.DMA for _ in range(4)],
            [pltpu.SemaphoreType.DMA for _ in range(2)],
            [pltpu.SemaphoreType.DMA for _ in range(2)],
            pltpu.VMEM_SHARED((n_pad, w_cols), jnp.float32),
        ],
    )
    def k(tab_hbm, ad_hbm, src_hbm, dst_hbm, out_hbm,
          tab_v, si, di, adv, wv, zb, tsem, isem, asem, ssem, acc_s):
        cid = lax.axis_index("c")
        sid = lax.axis_index("s")
        wid = sid * N_CORES + cid
        iota = lax.iota(jnp.int32, L)
        zeros = jnp.zeros((L,), jnp.float32)

        def chunk_base(j):
            return (wid * per_w + j) * CH

        def issue_idx(j, u):
            base = chunk_base(j)
            pltpu.async_copy(src_hbm.at[pl.ds(base, CH)], si[u], isem[u])
            pltpu.async_copy(dst_hbm.at[pl.ds(base, CH)], di[u], isem[u])

        def wait_idx(j, u):
            base = chunk_base(j)
            pltpu.make_async_copy(src_hbm.at[pl.ds(base, CH)], si[u],
                                  isem[u]).wait()
            pltpu.make_async_copy(dst_hbm.at[pl.ds(base, CH)], di[u],
                                  isem[u]).wait()

        def issue_ad(u, u2):
            pltpu.async_copy(ad_hbm.at[di[u]], adv[u2], asem[u2])

        def wait_ad(u, u2):
            pltpu.make_async_copy(ad_hbm.at[di[u]], adv[u2], asem[u2]).wait()

        def issue_scatter(u, u2):
            pltpu.async_copy(wv[u2], acc_s.at[di[u]], ssem[u2], add=True)

        def wait_scatter(u, u2):
            pltpu.make_async_copy(wv[u2], acc_s.at[di[u]], ssem[u2]).wait()

        def compute(u, u2):
            for g in range(CH // L):
                ridx = iota + g * L
                sv = si[u][pl.ds(g * L, L)]
                adv16 = adv[u2][pl.ds(g * L, L)]
                asv = plsc.load_gather(tab_v, [jnp.full((L,), d, jnp.int32), sv])
                logit = asv + adv16
                logit = jnp.where(logit >= 0.0, logit, logit * 0.2)
                w = jnp.exp(logit)
                for col in range(d):
                    hv = plsc.load_gather(
                        tab_v, [jnp.full((L,), col, jnp.int32), sv])
                    plsc.store_scatter(
                        wv[u2], [ridx, jnp.full((L,), col, jnp.int32)], w * hv)
                plsc.store_scatter(
                    wv[u2], [ridx, jnp.full((L,), d, jnp.int32)], w)

        tab_cp = pltpu.async_copy(tab_hbm, tab_v, tsem)
        issue_idx(0, 0)
        issue_idx(1, 1)

        def zrow(r, carry):
            ridx = iota + r * L
            for col in range(w_cols):
                plsc.store_scatter(
                    zb, [ridx, jnp.full((L,), col, jnp.int32)], zeros)
            return carry

        lax.fori_loop(0, 10, zrow, 0)

        def zcp(b, carry):
            pltpu.sync_copy(zb, acc_s.at[pl.ds(sid * rows_per_sub + b * 160, 160)])
            return carry

        lax.fori_loop(0, rows_per_sub // 160, zcp, 0)

        wait_idx(0, 0)
        issue_ad(0, 0)
        tab_cp.wait()
        plsc.subcore_barrier()

        def one_iter(j, u, u2, wait_s, pre_idx, pre_ad):
            wait_ad(u, u2)
            if wait_s:
                wait_scatter(u, u2)
            if pre_idx:
                issue_idx(j + 2, (u + 2) % 4)
            if pre_ad:
                wait_idx(j + 1, (u + 1) % 4)
                issue_ad((u + 1) % 4, (u2 + 1) % 2)
            compute(u, u2)
            issue_scatter(u, u2)

        for j in range(4):
            one_iter(j, j, j % 2, j >= 2, True, True)

        def loop_body(b, carry):
            j0 = b * 4
            for u in range(4):
                one_iter(j0 + u, u, u % 2, True, True, True)
            return carry

        lax.fori_loop(1, per_w // 4, loop_body, 0)

        one_iter(per_w - 2, (per_w - 2) % 4, (per_w - 2) % 2, True, False, True)
        one_iter(per_w - 1, (per_w - 1) % 4, (per_w - 1) % 2, True, False, False)
        wait_scatter((per_w - 2) % 4, (per_w - 2) % 2)
        wait_scatter((per_w - 1) % 4, (per_w - 1) % 2)

        if rem:
            @pl.when(wid < rem)
            def _():
                base = (n_workers * per_w + wid) * CH
                pltpu.async_copy(src_hbm.at[pl.ds(base, CH)], si[0], isem[0])
                pltpu.async_copy(dst_hbm.at[pl.ds(base, CH)], di[0], isem[0])
                pltpu.make_async_copy(src_hbm.at[pl.ds(base, CH)], si[0],
                                      isem[0]).wait()
                pltpu.make_async_copy(dst_hbm.at[pl.ds(base, CH)], di[0],
                                      isem[0]).wait()
                issue_ad(0, 0)
                wait_ad(0, 0)
                compute(0, 0)
                issue_scatter(0, 0)
                wait_scatter(0, 0)

        plsc.subcore_barrier()

        def ocp(b, carry):
            r0 = sid * rows_per_sub + b * 160
            pltpu.sync_copy(acc_s.at[pl.ds(r0, 160)], zb)
            pltpu.sync_copy(zb, out_hbm.at[cid, pl.ds(r0, 160)])
            return carry

        lax.fori_loop(0, rows_per_sub // 160, ocp, 0)

    return k(tableT, advec, src, dst)


def kernel(x, edge_index, W1, a1_src, a1_dst, b1, W2, a2_src, a2_dst, b2):
    n = x.shape[0]
    e = edge_index.shape[1]
    d = W1.shape[1]
    c = W2.shape[1]
    src = edge_index[0].astype(jnp.int32)
    dst = edge_index[1].astype(jnp.int32)

    t1, ad1 = _dense_first(x.T, W1.T, a1_src.reshape(1, d), a1_dst.reshape(1, d))
    parts1 = _edge_pass(t1, ad1.reshape(n), src, dst, d=d, n=n, e=e)
    t2, ad2 = _norm_dense_second(parts1, W2.T, a2_src.reshape(1, c),
                                 a2_dst.reshape(1, c), b1.reshape(1, d), n)
    parts2 = _edge_pass(t2, ad2.reshape(n), src, dst, d=c, n=n, e=e)
    return _norm_logsoftmax(parts2, b2.reshape(1, c), n)


# speedup vs baseline: 78.7170x; 1.0060x over previous
import functools

import jax
import jax.numpy as jnp
from jax import lax
from jax.experimental import pallas as pl
from jax.experimental.pallas import tpu as pltpu
from jax.experimental.pallas import tpu_sc as plsc

L = 16
N_CORES = 2
N_SUB = 16
CH = 128


def _dense_first(xT, W1T, a1s, a1d):
    n = xT.shape[1]
    d = W1T.shape[0]

    def body(x_ref, w_ref, s_ref, t_ref, o_ref, ad_ref):
        hT = jnp.dot(w_ref[...], x_ref[...], preferred_element_type=jnp.float32)
        asT = jnp.dot(s_ref[...], hT, preferred_element_type=jnp.float32)
        adT = jnp.dot(t_ref[...], hT, preferred_element_type=jnp.float32)
        o_ref[...] = jnp.concatenate([hT, asT], axis=0)
        ad_ref[...] = adT

    return pl.pallas_call(
        body,
        out_shape=[jax.ShapeDtypeStruct((d + 1, n), jnp.float32),
                   jax.ShapeDtypeStruct((1, n), jnp.float32)],
    )(xT, W1T, a1s, a1d)


def _norm_dense_second(parts, W2tr, a2s, a2d, b1, n):
    c, d = W2tr.shape

    def body(p_ref, w_ref, s_ref, t_ref, b_ref, o_ref, ad_ref):
        acc = p_ref[0, :n] + p_ref[1, :n]
        ssum = acc[:, d:d + 1]
        h = jnp.where(ssum > 0.0, acc[:, :d] / ssum, 0.0) + b_ref[...]
        h = jnp.maximum(h, 0.0)
        h2T = lax.dot_general(w_ref[...], h, (((1,), (1,)), ((), ())),
                              preferred_element_type=jnp.float32)
        asT = jnp.dot(s_ref[...], h2T, preferred_element_type=jnp.float32)
        adT = jnp.dot(t_ref[...], h2T, preferred_element_type=jnp.float32)
        o_ref[...] = jnp.concatenate([h2T, asT], axis=0)
        ad_ref[...] = adT

    return pl.pallas_call(
        body,
        out_shape=[jax.ShapeDtypeStruct((c + 1, n), jnp.float32),
                   jax.ShapeDtypeStruct((1, n), jnp.float32)],
    )(parts, W2tr, a2s, a2d, b1)


def _norm_logsoftmax(parts, b2, n):
    c = b2.shape[1]

    def body(p_ref, b_ref, o_ref):
        acc = p_ref[0, :n] + p_ref[1, :n]
        ssum = acc[:, c:c + 1]
        o = jnp.where(ssum > 0.0, acc[:, :c] / ssum, 0.0) + b_ref[...]
        m = jnp.max(o, axis=1, keepdims=True)
        z = o - m
        o_ref[...] = z - jnp.log(jnp.sum(jnp.exp(z), axis=1, keepdims=True))

    return pl.pallas_call(
        body,
        out_shape=jax.ShapeDtypeStruct((n, c), jnp.float32),
    )(parts, b2)


@functools.partial(jax.jit, static_argnames=("d", "n", "e"))
def _edge_pass(tableT, advec, src, dst, *, d, n, e):
    w_cols = d + 1
    n_workers = N_CORES * N_SUB
    rows_per_sub = ((n + N_SUB - 1) // N_SUB + 159) // 160 * 160
    n_pad = N_SUB * rows_per_sub
    n_chunks = e // CH
    per_w = n_chunks // n_workers
    rem = n_chunks % n_workers
    assert per_w % 4 == 2 and per_w >= 6, "pipeline peels 4 + tail 2"

    mesh = plsc.VectorSubcoreMesh(core_axis_name="c", subcore_axis_name="s")

    @functools.partial(
        pl.kernel,
        out_type=jax.ShapeDtypeStruct((N_CORES, n_pad, w_cols), jnp.float32),
        mesh=mesh,
        compiler_params=pltpu.CompilerParams(
            needs_layout_passes=False, use_tc_tiling_on_sc=False),
        scratch_types=[
            pltpu.VMEM((d + 1, n), jnp.float32),
            [pltpu.VMEM((CH,), jnp.int32) for _ in range(4)],
            [pltpu.VMEM((CH,), jnp.int32) for _ in range(4)],
            [pltpu.VMEM((CH,), jnp.float32) for _ in range(2)],
            [pltpu.VMEM((CH, w_cols), jnp.float32) for _ in range(2)],
            pltpu.VMEM((160, w_cols), jnp.float32),
            pltpu.SemaphoreType.DMA,
            [pltpu.SemaphoreType.DMA for _ in range(4)],
            [pltpu.SemaphoreType.DMA for _ in range(2)],
            [pltpu.SemaphoreType.DMA for _ in range(2)],
            pltpu.VMEM_SHARED((n_pad, w_cols), jnp.float32),
        ],
    )
    def k(tab_hbm, ad_hbm, src_hbm, dst_hbm, out_hbm,
          tab_v, si, di, adv, wv, zb, tsem, isem, asem, ssem, acc_s):
        cid = lax.axis_index("c")
        sid = lax.axis_index("s")
        wid = sid * N_CORES + cid
        iota = lax.iota(jnp.int32, L)
        zeros = jnp.zeros((L,), jnp.float32)

        def chunk_base(j):
            return (wid * per_w + j) * CH

        def issue_idx(j, u):
            base = chunk_base(j)
            pltpu.async_copy(src_hbm.at[pl.ds(base, CH)], si[u], isem[u])
            pltpu.async_copy(dst_hbm.at[pl.ds(base, CH)], di[u], isem[u])

        def wait_idx(j, u):
            base = chunk_base(j)
            pltpu.make_async_copy(src_hbm.at[pl.ds(base, CH)], si[u],
                                  isem[u]).wait()
            pltpu.make_async_copy(dst_hbm.at[pl.ds(base, CH)], di[u],
                                  isem[u]).wait()

        def issue_ad(u, u2):
            pltpu.async_copy(ad_hbm.at[di[u]], adv[u2], asem[u2])

        def wait_ad(u, u2):
            pltpu.make_async_copy(ad_hbm.at[di[u]], adv[u2], asem[u2]).wait()

        def issue_scatter(u, u2):
            pltpu.async_copy(wv[u2], acc_s.at[di[u]], ssem[u2], add=True)

        def wait_scatter(u, u2):
            pltpu.make_async_copy(wv[u2], acc_s.at[di[u]], ssem[u2]).wait()

        def compute(u, u2):
            for g in range(CH // L):
                ridx = iota + g * L
                sv = si[u][pl.ds(g * L, L)]
                adv16 = adv[u2][pl.ds(g * L, L)]
                asv = plsc.load_gather(tab_v, [jnp.full((L,), d, jnp.int32), sv])
                logit = asv + adv16
                logit = jnp.where(logit >= 0.0, logit, logit * 0.2)
                w = jnp.exp(logit)
                for col in range(d):
                    hv = plsc.load_gather(
                        tab_v, [jnp.full((L,), col, jnp.int32), sv])
                    plsc.store_scatter(
                        wv[u2], [ridx, jnp.full((L,), col, jnp.int32)], w * hv)
                plsc.store_scatter(
                    wv[u2], [ridx, jnp.full((L,), d, jnp.int32)], w)

        tab_cp = pltpu.async_copy(tab_hbm, tab_v, tsem)
        issue_idx(0, 0)
        issue_idx(1, 1)

        def zrow(r, carry):
            ridx = iota + r * L
            for col in range(w_cols):
                plsc.store_scatter(
                    zb, [ridx, jnp.full((L,), col, jnp.int32)], zeros)
            return carry

        lax.fori_loop(0, 10, zrow, 0)

        def zcp(b, carry):
            pltpu.sync_copy(zb, acc_s.at[pl.ds(sid * rows_per_sub + b * 160, 160)])
            return carry

        lax.fori_loop(0, rows_per_sub // 160, zcp, 0)

        wait_idx(0, 0)
        issue_ad(0, 0)
        tab_cp.wait()
        plsc.subcore_barrier()

        def one_iter(j, u, u2, wait_s, pre_idx, pre_ad):
            wait_ad(u, u2)
            if wait_s:
                wait_scatter(u, u2)
            if pre_idx:
                issue_idx(j + 2, (u + 2) % 4)
            if pre_ad:
                wait_idx(j + 1, (u + 1) % 4)
                issue_ad((u + 1) % 4, (u2 + 1) % 2)
            compute(u, u2)
            issue_scatter(u, u2)

        for j in range(4):
            one_iter(j, j, j % 2, j >= 2, True, True)

        def loop_body(b, carry):
            j0 = b * 4
            for u in range(4):
                one_iter(j0 + u, u, u % 2, True, True, True)
            return carry

        lax.fori_loop(1, per_w // 4, loop_body, 0)

        one_iter(per_w - 2, (per_w - 2) % 4, (per_w - 2) % 2, True, False, True)
        one_iter(per_w - 1, (per_w - 1) % 4, (per_w - 1) % 2, True, False, False)
        wait_scatter((per_w - 2) % 4, (per_w - 2) % 2)
        wait_scatter((per_w - 1) % 4, (per_w - 1) % 2)

        if rem:
            @pl.when(wid < rem)
            def _():
                base = (n_workers * per_w + wid) * CH
                pltpu.async_copy(src_hbm.at[pl.ds(base, CH)], si[0], isem[0])
                pltpu.async_copy(dst_hbm.at[pl.ds(base, CH)], di[0], isem[0])
                pltpu.make_async_copy(src_hbm.at[pl.ds(base, CH)], si[0],
                                      isem[0]).wait()
                pltpu.make_async_copy(dst_hbm.at[pl.ds(base, CH)], di[0],
                                      isem[0]).wait()
                issue_ad(0, 0)
                wait_ad(0, 0)
                compute(0, 0)
                issue_scatter(0, 0)
                wait_scatter(0, 0)

        plsc.subcore_barrier()

        r0 = sid * rows_per_sub
        pltpu.sync_copy(acc_s.at[pl.ds(r0, rows_per_sub)],
                        out_hbm.at[cid, pl.ds(r0, rows_per_sub)])

    return k(tableT, advec, src, dst)


def kernel(x, edge_index, W1, a1_src, a1_dst, b1, W2, a2_src, a2_dst, b2):
    n = x.shape[0]
    e = edge_index.shape[1]
    d = W1.shape[1]
    c = W2.shape[1]
    src = edge_index[0].astype(jnp.int32)
    dst = edge_index[1].astype(jnp.int32)

    t1, ad1 = _dense_first(x.T, W1.T, a1_src.reshape(1, d), a1_dst.reshape(1, d))
    parts1 = _edge_pass(t1, ad1.reshape(n), src, dst, d=d, n=n, e=e)
    t2, ad2 = _norm_dense_second(parts1, W2.T, a2_src.reshape(1, c),
                                 a2_dst.reshape(1, c), b1.reshape(1, d), n)
    parts2 = _edge_pass(t2, ad2.reshape(n), src, dst, d=c, n=n, e=e)
    return _norm_logsoftmax(parts2, b2.reshape(1, c), n)
